# Initial kernel scaffold; baseline (speedup 1.0000x reference)
import jax
import jax.numpy as jnp
from jax.experimental import pallas as pl


def kernel(node_features, edge_index, distance, Wi1, bi1, Wi2, bi2, We1, be1, We2, be2, Wn1, bn1, Wn2, bn2, Wc1, bc1, Wc2, bc2):
    raise NotImplementedError("write your pallas kernel here")


# speedup vs baseline: 2.0890x; 2.0890x over previous
import functools

import jax
import jax.numpy as jnp
from jax import lax
from jax.experimental import pallas as pl
from jax.experimental.pallas import tpu as pltpu
from jax.experimental.pallas import tpu_sc as plsc

N = 10000
E = 160000
NACC = 10240
NC, NS = 2, 16
NW = NC * NS
C = 128
K = 40
EW = K * C
EP = NW * EW
DUMMY = N
NROWS = NACC // NS


def _sc_mesh():
    return plsc.VectorSubcoreMesh(
        core_axis_name="c", subcore_axis_name="s", num_cores=NC, num_subcores=NS)


@functools.cache
def _make_sc_gather(D):
    @functools.partial(
        pl.kernel,
        mesh=_sc_mesh(),
        out_type=jax.ShapeDtypeStruct((EP, D), jnp.float32),
        scratch_types=[
            pltpu.VMEM((C,), jnp.int32),
            pltpu.VMEM((C,), jnp.int32),
            pltpu.VMEM((C, D), jnp.float32),
            pltpu.VMEM((C, D), jnp.float32),
            pltpu.SemaphoreType.DMA,
            pltpu.SemaphoreType.DMA,
        ],
        name=f"sc_gather_add_{D}",
    )
    def k(gd_hbm, gs_hbm, dsti_hbm, srci_hbm, out_hbm,
          dst_c, src_c, bufd, bufs, semd, sems):
        wid = lax.axis_index("s") * NC + lax.axis_index("c")
        base = wid * EW

        def body(j, _):
            row = wid * K + j
            pltpu.sync_copy(dsti_hbm.at[row], dst_c)
            pltpu.sync_copy(srci_hbm.at[row], src_c)
            cpd = pltpu.async_copy(gd_hbm.at[dst_c], bufd, semd)
            cps = pltpu.async_copy(gs_hbm.at[src_c], bufs, sems)
            cpd.wait()
            cps.wait()

            def addrow(r, _):
                for u in range(D // 16):
                    sl = pl.ds(u * 16, 16)
                    bufd[r, sl] = bufd[r, sl] + bufs[r, sl]
                return 0

            lax.fori_loop(0, C, addrow, 0)
            pltpu.sync_copy(bufd, out_hbm.at[pl.ds(base + j * C, C)])
            return 0

        lax.fori_loop(0, K, body, 0)

    return k


def _sc_gather128(Gd, Gs, dst_g, src_g):
    return _make_sc_gather(128)(Gd, Gs, dst_g, src_g)


@functools.cache
def _make_sc_scatter():
    @functools.partial(
        pl.kernel,
        mesh=_sc_mesh(),
        out_type=jax.ShapeDtypeStruct((NC, NACC, 128), jnp.float32),
        scratch_types=[
            pltpu.VMEM((C,), jnp.int32),
            pltpu.VMEM((C, 128), jnp.float32),
            pltpu.VMEM_SHARED((NACC, 128), jnp.float32),
            pltpu.SemaphoreType.DMA,
        ],
        name="sc_scatter_add",
    )
    def k(ehid_hbm, dsti_hbm, zeros_hbm, parts_hbm, dst_c, buf, acc, sem):
        cid = lax.axis_index("c")
        sid = lax.axis_index("s")
        wid = sid * NC + cid
        rows = pl.ds(sid * NROWS, NROWS)
        pltpu.sync_copy(zeros_hbm.at[rows], acc.at[rows])
        plsc.subcore_barrier()
        base = wid * EW

        def body(j, _):
            pltpu.sync_copy(dsti_hbm.at[wid * K + j], dst_c)
            pltpu.sync_copy(ehid_hbm.at[pl.ds(base + j * C, C)], buf)
            pltpu.sync_copy(buf, acc.at[dst_c], add=True)
            return 0

        lax.fori_loop(0, K, body, 0)
        plsc.subcore_barrier()
        pltpu.sync_copy(acc.at[rows], parts_hbm.at[cid].at[rows])

    return k


def _sc_scatter(ehid, dst_s, zeros_acc):
    return _make_sc_scatter()(ehid, dst_s, zeros_acc)


_BN = 640
_BE = 2048


def _full(shape):
    return pl.BlockSpec(shape, lambda i: tuple(0 for _ in shape))


def _nf2(nf, w):
    return nf[:, 0:1] * w[0:1, :] + nf[:, 1:2] * w[1:2, :]


def _init_body(nf_ref, wi1, bi1, wi2, bi2, wghd, wgfd, be1, wghs, wgfs,
               h_ref, gd_ref, gs_ref):
    nf = nf_ref[...]
    t = jnp.maximum(_nf2(nf, wi1[...]) + bi1[...], 0.0)
    h = jnp.maximum(
        jnp.dot(t, wi2[...], preferred_element_type=jnp.float32) + bi2[...], 0.0)
    h_ref[...] = h
    gd_ref[...] = (jnp.dot(h, wghd[...], preferred_element_type=jnp.float32)
                   + _nf2(nf, wgfd[...]) + be1[...])
    gs_ref[...] = (jnp.dot(h, wghs[...], preferred_element_type=jnp.float32)
                   + _nf2(nf, wgfs[...]))


def _tc_init(nfp, wi1, bi1, wi2, bi2, wghd, wgfd, be1, wghs, wgfs):
    grid = (NACC // _BN,)
    blk = lambda c: pl.BlockSpec((_BN, c), lambda i: (i, 0))
    return pl.pallas_call(
        _init_body,
        grid=grid,
        in_specs=[blk(2), _full((2, 128)), _full((1, 128)), _full((128, 64)),
                  _full((1, 64)), _full((64, 128)), _full((2, 128)),
                  _full((1, 128)), _full((64, 128)), _full((2, 128))],
        out_specs=[blk(64), blk(128), blk(128)],
        out_shape=[jax.ShapeDtypeStruct((NACC, 64), jnp.float32),
                   jax.ShapeDtypeStruct((NACC, 128), jnp.float32),
                   jax.ShapeDtypeStruct((NACC, 128), jnp.float32)],
    )(nfp, wi1, bi1, wi2, bi2, wghd, wgfd, be1, wghs, wgfs)


def _edge_body(pre_ref, d_ref, wd, we2, be2, out_ref):
    p = jnp.maximum(pre_ref[...] + d_ref[...] * wd[...], 0.0)
    e = jnp.maximum(
        jnp.dot(p, we2[...], preferred_element_type=jnp.float32) + be2[...], 0.0)
    out_ref[...] = jnp.concatenate([e, jnp.zeros_like(e)], axis=1)


def _tc_edge(pre, dist2, wd, we2, be2):
    grid = (EP // _BE,)
    return pl.pallas_call(
        _edge_body,
        grid=grid,
        in_specs=[pl.BlockSpec((_BE, 128), lambda i: (i, 0)),
                  pl.BlockSpec((_BE, 1), lambda i: (i, 0)),
                  _full((1, 128)), _full((128, 64)), _full((1, 64))],
        out_specs=pl.BlockSpec((_BE, 128), lambda i: (i, 0)),
        out_shape=jax.ShapeDtypeStruct((EP, 128), jnp.float32),
    )(pre, dist2, wd, we2, be2)


def _node_body(h_ref, nf_ref, pa_ref, pb_ref, wnh, wnf, wna, bn1, wn2, bn2,
               wghd, wgfd, be1, wghs, wgfs, hn_ref, gd_ref, gs_ref):
    agg = pa_ref[0][:, 0:64] + pb_ref[0][:, 0:64]
    h = h_ref[...]
    nf = nf_ref[...]
    t = (jnp.dot(h, wnh[...], preferred_element_type=jnp.float32)
         + _nf2(nf, wnf[...])
         + jnp.dot(agg, wna[...], preferred_element_type=jnp.float32))
    t = jnp.maximum(t + bn1[...], 0.0)
    hn = jnp.maximum(
        jnp.dot(t, wn2[...], preferred_element_type=jnp.float32) + bn2[...], 0.0)
    hn_ref[...] = hn
    gd_ref[...] = (jnp.dot(hn, wghd[...], preferred_element_type=jnp.float32)
                   + _nf2(nf, wgfd[...]) + be1[...])
    gs_ref[...] = (jnp.dot(hn, wghs[...], preferred_element_type=jnp.float32)
                   + _nf2(nf, wgfs[...]))


def _tc_node(h, nfp, parts, wnh, wnf, wna, bn1, wn2, bn2,
             wghd, wgfd, be1, wghs, wgfs):
    grid = (NACC // _BN,)
    blk = lambda c: pl.BlockSpec((_BN, c), lambda i: (i, 0))
    part = lambda s: pl.BlockSpec((1, _BN, 128), lambda i, _s=s: (_s, i, 0))
    return pl.pallas_call(
        _node_body,
        grid=grid,
        in_specs=[blk(64), blk(2), part(0), part(1),
                  _full((64, 128)), _full((2, 128)), _full((64, 128)),
                  _full((1, 128)), _full((128, 64)), _full((1, 64)),
                  _full((64, 128)), _full((2, 128)), _full((1, 128)),
                  _full((64, 128)), _full((2, 128))],
        out_specs=[blk(64), blk(128), blk(128)],
        out_shape=[jax.ShapeDtypeStruct((NACC, 64), jnp.float32),
                   jax.ShapeDtypeStruct((NACC, 128), jnp.float32),
                   jax.ShapeDtypeStruct((NACC, 128), jnp.float32)],
    )(h, nfp, parts, parts, wnh, wnf, wna, bn1, wn2, bn2,
      wghd, wgfd, be1, wghs, wgfs)


def _proj_body(h_ref, nf_ref, whd, wfd, bc1, whs, wfs, kd_ref, ks_ref):
    h = h_ref[...]
    nf = nf_ref[...]
    kd_ref[...] = (jnp.dot(h, whd[...], preferred_element_type=jnp.float32)
                   + _nf2(nf, wfd[...]) + bc1[...])
    ks_ref[...] = (jnp.dot(h, whs[...], preferred_element_type=jnp.float32)
                   + _nf2(nf, wfs[...]))


def _tc_proj(h, nfp, whd, wfd, bc1, whs, wfs):
    grid = (NACC // _BN,)
    blk = lambda c: pl.BlockSpec((_BN, c), lambda i: (i, 0))
    return pl.pallas_call(
        _proj_body,
        grid=grid,
        in_specs=[blk(64), blk(2), _full((64, 128)), _full((2, 128)),
                  _full((1, 128)), _full((64, 128)), _full((2, 128))],
        out_specs=[blk(128), blk(128)],
        out_shape=[jax.ShapeDtypeStruct((NACC, 128), jnp.float32),
                   jax.ShapeDtypeStruct((NACC, 128), jnp.float32)],
    )(h, nfp, whd, wfd, bc1, whs, wfs)


def _head_body(pre_ref, d_ref, wcd, wc2r, bc2, out_ref):
    c = jnp.maximum(pre_ref[:, 0:64] + d_ref[...] * wcd[...], 0.0)
    z = jnp.sum(c * wc2r[...], axis=1, keepdims=True) + bc2[...]
    out_ref[...] = jax.nn.sigmoid(z)


def _tc_head(pre_c, dist2, wcd, wc2r, bc2):
    grid = (EP // _BE,)
    return pl.pallas_call(
        _head_body,
        grid=grid,
        in_specs=[pl.BlockSpec((_BE, 128), lambda i: (i, 0)),
                  pl.BlockSpec((_BE, 1), lambda i: (i, 0)),
                  _full((1, 64)), _full((1, 64)), _full((1, 1))],
        out_specs=pl.BlockSpec((_BE, 1), lambda i: (i, 0)),
        out_shape=jax.ShapeDtypeStruct((EP, 1), jnp.float32),
    )(pre_c, dist2, wcd, wc2r, bc2)


def kernel(node_features, edge_index, distance, Wi1, bi1, Wi2, bi2,
           We1, be1, We2, be2, Wn1, bn1, Wn2, bn2, Wc1, bc1, Wc2, bc2):
    f32 = jnp.float32
    src = edge_index[0].astype(jnp.int32)
    dst = edge_index[1].astype(jnp.int32)
    pad = EP - E
    src_g = jnp.concatenate([src, jnp.zeros((pad,), jnp.int32)]).reshape(NW * K, C)
    dst_g = jnp.concatenate([dst, jnp.zeros((pad,), jnp.int32)]).reshape(NW * K, C)
    dst_s = jnp.concatenate(
        [dst, jnp.full((pad,), DUMMY, jnp.int32)]).reshape(NW * K, C)
    dist2 = jnp.concatenate([distance, jnp.zeros((pad,), f32)])[:, None]
    nfp = jnp.zeros((NACC, 2), f32).at[:N].set(node_features)
    zeros_acc = jnp.zeros((NACC, 128), f32)

    wgfd, wghd = We1[0:2], We1[2:66]
    wgfs, wghs = We1[66:68], We1[68:132]
    wd = We1[132:133]
    pad64 = lambda w: jnp.pad(w, ((0, 0), (0, 64)))
    wcfd, wchd = pad64(Wc1[0:2]), pad64(Wc1[2:66])
    wcfs, wchs = pad64(Wc1[66:68]), pad64(Wc1[68:132])
    wcd = Wc1[132:133]
    wnh, wnf, wna = Wn1[0:64], Wn1[64:66], Wn1[66:130]
    bi1r, bi2r = bi1[None, :], bi2[None, :]
    be1r, be2r = be1[None, :], be2[None, :]
    bn1r, bn2r = bn1[None, :], bn2[None, :]
    bc1r, bc2r = jnp.pad(bc1[None, :], ((0, 0), (0, 64))), bc2[None, :]
    wc2r = Wc2.T

    h, Gd, Gs = _tc_init(nfp, Wi1, bi1r, Wi2, bi2r, wghd, wgfd, be1r, wghs, wgfs)

    def it(_, carry):
        h, Gd, Gs = carry
        pre = _sc_gather128(Gd, Gs, dst_g, src_g)
        ehid = _tc_edge(pre, dist2, wd, We2, be2r)
        parts = _sc_scatter(ehid, dst_s, zeros_acc)
        h, Gd, Gs = _tc_node(h, nfp, parts, wnh, wnf, wna, bn1r, Wn2, bn2r,
                             wghd, wgfd, be1r, wghs, wgfs)
        return (h, Gd, Gs)

    h, Gd, Gs = lax.fori_loop(0, 16, it, (h, Gd, Gs))

    Kd, Ks = _tc_proj(h, nfp, wchd, wcfd, bc1r, wchs, wcfs)
    pre_c = _sc_gather128(Kd, Ks, dst_g, src_g)
    out = _tc_head(pre_c, dist2, wcd, wc2r, bc2r)
    return out[:E]
